# baseline (speedup 1.0000x reference)
import functools

import jax
import jax.numpy as jnp
from jax import lax
from jax.experimental import pallas as pl
from jax.experimental.pallas import tpu as pltpu
from jax.experimental.pallas import tpu_sc as plsc

B = 4096
H = 200
D = 64
LANES = 16
NC, NS = 2, 16
C = 256
K = C // 128
NCOL = B // C
TPT = H // 2

_mesh = plsc.VectorSubcoreMesh(core_axis_name="c", subcore_axis_name="s")

_cp = pltpu.CompilerParams(needs_layout_passes=False, use_tc_tiling_on_sc=False)


@functools.partial(
    pl.kernel,
    compiler_params=_cp,
    out_type=jax.ShapeDtypeStruct((H, D, B), jnp.float32),
    mesh=_mesh,
    scratch_types=[
        pltpu.VMEM((K, 128), jnp.int32),
        pltpu.VMEM((K, 128), jnp.int32),
        pltpu.VMEM((C, D), jnp.float32),
        pltpu.VMEM((C, D), jnp.float32),
        pltpu.VMEM((D, C), jnp.float32),
        pltpu.VMEM((D, C), jnp.float32),
        pltpu.SemaphoreType.DMA,
        pltpu.SemaphoreType.DMA,
        pltpu.SemaphoreType.DMA,
        pltpu.SemaphoreType.DMA,
    ],
)
def _embed_lookup(table_hbm, idx_hbm, out_hbm,
                  idx0, idx1, rows0, rows1, tr0, tr1,
                  sg0, sg1, so0, so1):
    wid = lax.axis_index("s") * NC + lax.axis_index("c")
    b0 = (wid % NCOL) * C
    h_base = wid // NCOL

    def load_and_fire(t, idx_v, rows_v, sem):
        h = h_base + 2 * t
        for j in range(K):
            pltpu.sync_copy(idx_hbm.at[h, pl.ds(b0 + j * 128, 128)],
                            idx_v.at[j])
        for j in range(K):
            pltpu.async_copy(
                table_hbm.at[idx_v.at[j]],
                rows_v.at[pl.ds(j * 128, 128)],
                sem,
            )

    def drain_gather(rows_v, sem):
        pltpu.make_async_copy(table_hbm.at[pl.ds(0, C)], rows_v, sem).wait()

    def fixup(idx_v, rows_v):
        acc = idx_v[0, pl.ds(0, LANES)]
        for g in range(1, C // LANES):
            acc = jnp.minimum(acc, idx_v[g // 8, pl.ds((g % 8) * LANES, LANES)])

        @pl.when(jnp.min(acc) == 0)
        def _():
            zeros = jnp.zeros((LANES,), jnp.float32)
            for g in range(C // LANES):
                vec = idx_v[g // 8, pl.ds((g % 8) * LANES, LANES)]

                @pl.when(jnp.min(vec) == 0)
                def _():
                    mask = vec == 0
                    row_ids = lax.iota(jnp.int32, LANES) + (g * LANES)

                    @pl.loop(0, D)
                    def _(col):
                        col_ids = jnp.full((LANES,), 0, jnp.int32) + col
                        plsc.store_scatter(rows_v, [row_ids, col_ids],
                                           zeros, mask=mask)

    def transpose(rows_v, tr_v):
        @pl.loop(0, C // LANES)
        def _(g):
            colbase = g * LANES
            b_ids = lax.iota(jnp.int32, LANES) + colbase
            for d in range(D):
                d_ids = jnp.full((LANES,), d, jnp.int32)
                vals = plsc.load_gather(rows_v, [b_ids, d_ids])
                tr_v.at[d, pl.ds(colbase, LANES)][...] = vals

    def fire_out(t, tr_v, sem):
        h = h_base + 2 * t
        pltpu.async_copy(tr_v, out_hbm.at[h, :, pl.ds(b0, C)], sem)

    def wait_out(tr_v, sem):
        pltpu.make_async_copy(tr_v, out_hbm.at[0, :, pl.ds(b0, C)],
                              sem).wait()

    load_and_fire(0, idx0, rows0, sg0)

    @pl.loop(0, TPT // 2)
    def _(tt):
        ta = 2 * tt
        tb = ta + 1
        load_and_fire(tb, idx1, rows1, sg1)
        drain_gather(rows0, sg0)
        fixup(idx0, rows0)

        @pl.when(tt > 0)
        def _():
            wait_out(tr0, so0)

        transpose(rows0, tr0)
        fire_out(ta, tr0, so0)

        @pl.when(tb + 1 < TPT)
        def _():
            load_and_fire(tb + 1, idx0, rows0, sg0)

        drain_gather(rows1, sg1)
        fixup(idx1, rows1)

        @pl.when(tt > 0)
        def _():
            wait_out(tr1, so1)

        transpose(rows1, tr1)
        fire_out(tb, tr1, so1)

    wait_out(tr0, so0)
    wait_out(tr1, so1)


def kernel(x, table):
    idx = x.astype(jnp.int32).T
    out = _embed_lookup(table, idx)
    return jnp.transpose(out, (2, 0, 1))
